# Initial kernel scaffold; baseline (speedup 1.0000x reference)
import jax
import jax.numpy as jnp
from jax.experimental import pallas as pl


def kernel(z, edge_index, params):
    raise NotImplementedError("write your pallas kernel here")


# speedup vs baseline: 18.5521x; 18.5521x over previous
import functools

import jax
import jax.numpy as jnp
from jax import lax
from jax.experimental import pallas as pl
from jax.experimental.pallas import tpu as pltpu
from jax.experimental.pallas import tpu_sc as plsc

B = 64
NUM_NODES = 196
N = B * NUM_NODES
E = 401408
LATENT = 128
NEG = 0.2

NUM_TILES = 32
EPT = E // NUM_TILES
CHUNK = 128
NCHUNK = EPT // CHUNK
ROWBLK = 896


def _leaky(x):
    return jnp.where(x > 0, x, NEG * x)


def _fc_body(z_ref, w_ref, b_ref, o_ref):
    acc = lax.dot_general(z_ref[...], w_ref[...],
                          (((1,), (1,)), ((), ())),
                          preferred_element_type=jnp.float32)
    o_ref[...] = jnp.maximum(acc + b_ref[...], 0.0)


def _fc(z, fc_w, fc_b):
    nblk = 49
    blk = (NUM_NODES * LATENT) // nblk
    return pl.pallas_call(
        _fc_body,
        grid=(nblk,),
        in_specs=[
            pl.BlockSpec((B, LATENT), lambda i: (0, 0)),
            pl.BlockSpec((blk, LATENT), lambda i: (i, 0)),
            pl.BlockSpec((1, blk), lambda i: (0, i)),
        ],
        out_specs=pl.BlockSpec((B, blk), lambda i: (0, i)),
        out_shape=jax.ShapeDtypeStruct((B, NUM_NODES * LATENT), jnp.float32),
    )(z, fc_w, fc_b.reshape(1, -1))


def _pre_body(x_ref, w_ref, asrc_ref, adst_ref, h_ref, s_ref, d_ref, sm_ref):
    i = pl.program_id(0)
    h = lax.dot_general(x_ref[...], w_ref[...],
                        (((1,), (0,)), ((), ())),
                        preferred_element_type=jnp.float32)
    h_ref[...] = h
    s = jnp.sum(h * asrc_ref[...], axis=1, keepdims=True)
    d = jnp.sum(h * adst_ref[...], axis=1, keepdims=True)
    s_ref[...] = s
    d_ref[...] = d

    @pl.when(i == 0)
    def _():
        sm_ref[...] = jnp.full((1, 1), -jnp.inf, jnp.float32)

    sm_ref[...] = jnp.maximum(sm_ref[...], jnp.max(s))


def _pre(x, w, a_src, a_dst):
    nblk = N // ROWBLK
    return pl.pallas_call(
        _pre_body,
        grid=(nblk,),
        in_specs=[
            pl.BlockSpec((ROWBLK, LATENT), lambda i: (i, 0)),
            pl.BlockSpec((LATENT, LATENT), lambda i: (0, 0)),
            pl.BlockSpec((1, LATENT), lambda i: (0, 0)),
            pl.BlockSpec((1, LATENT), lambda i: (0, 0)),
        ],
        out_specs=[
            pl.BlockSpec((ROWBLK, LATENT), lambda i: (i, 0)),
            pl.BlockSpec((ROWBLK, 1), lambda i: (i, 0)),
            pl.BlockSpec((ROWBLK, 1), lambda i: (i, 0)),
            pl.BlockSpec((1, 1), lambda i: (0, 0)),
        ],
        out_shape=[
            jax.ShapeDtypeStruct((N, LATENT), jnp.float32),
            jax.ShapeDtypeStruct((N, 1), jnp.float32),
            jax.ShapeDtypeStruct((N, 1), jnp.float32),
            jax.ShapeDtypeStruct((1, 1), jnp.float32),
        ],
    )(x, w, a_src.reshape(1, -1), a_dst.reshape(1, -1))


def _combine_body(relu, p0_ref, p1_ref, den_ref, h_ref, s_ref, d_ref,
                  sm_ref, b_ref, o_ref):
    s = s_ref[...]
    d = d_ref[...]
    smax = sm_ref[0, 0]
    m = _leaky(smax + d)
    ea_self = jnp.exp(_leaky(s + d) - m)
    den_tot = jnp.sum(den_ref[...], axis=0)[:, None] + ea_self
    num = p0_ref[...] + p1_ref[...] + ea_self * h_ref[...]
    out = num / den_tot + b_ref[...]
    if relu:
        out = jnp.maximum(out, 0.0)
    o_ref[...] = out


def _combine(p0, p1, den, h, s, d, smax, bias, relu):
    nblk = N // ROWBLK
    return pl.pallas_call(
        functools.partial(_combine_body, relu),
        grid=(nblk,),
        in_specs=[
            pl.BlockSpec((ROWBLK, LATENT), lambda i: (i, 0)),
            pl.BlockSpec((ROWBLK, LATENT), lambda i: (i, 0)),
            pl.BlockSpec((NUM_TILES, ROWBLK), lambda i: (0, i)),
            pl.BlockSpec((ROWBLK, LATENT), lambda i: (i, 0)),
            pl.BlockSpec((ROWBLK, 1), lambda i: (i, 0)),
            pl.BlockSpec((ROWBLK, 1), lambda i: (i, 0)),
            pl.BlockSpec((1, 1), lambda i: (0, 0)),
            pl.BlockSpec((1, LATENT), lambda i: (0, 0)),
        ],
        out_specs=pl.BlockSpec((ROWBLK, LATENT), lambda i: (i, 0)),
        out_shape=jax.ShapeDtypeStruct((N, LATENT), jnp.float32),
    )(p0, p1, den, h, s, d, smax, bias.reshape(1, -1))


def _ew_body(s_hbm, d_hbm, sm_hbm, src_hbm, dst_hbm,
             ea_hbm, den_hbm,
             s_v, d_v, den_v, sm_v, src_v, dst_v, ea_v):
    c = lax.axis_index("c")
    sub = lax.axis_index("s")
    wid = c * 16 + sub
    base = wid * EPT

    pltpu.sync_copy(s_hbm, s_v)
    pltpu.sync_copy(d_hbm, d_v)
    pltpu.sync_copy(sm_hbm, sm_v)

    zero16 = jnp.zeros((16,), jnp.float32)

    def _zden(i, carry):
        den_v[pl.ds(i * 16, 16)] = zero16
        return carry
    lax.fori_loop(0, N // 16, _zden, 0)

    smax = sm_v[...]

    def _chunk(k, carry):
        off = base + k * CHUNK
        pltpu.sync_copy(src_hbm.at[pl.ds(off, CHUNK)], src_v)
        pltpu.sync_copy(dst_hbm.at[pl.ds(off, CHUNK)], dst_v)

        for j in range(CHUNK // 16):
            isrc = src_v[pl.ds(j * 16, 16)]
            idst = dst_v[pl.ds(j * 16, 16)]
            sv = plsc.load_gather(s_v, [isrc])
            dv = plsc.load_gather(d_v, [idst])
            t = smax + dv
            m = jnp.where(t > 0, t, NEG * t)
            a = sv + dv
            a = jnp.where(a > 0, a, NEG * a)
            ea = jnp.exp(a - m)
            plsc.addupdate_scatter(den_v, [idst], ea)
            ea_v[pl.ds(j * 16, 16)] = ea

        pltpu.sync_copy(ea_v, ea_hbm.at[pl.ds(off, CHUNK)])
        return carry

    lax.fori_loop(0, NCHUNK, _chunk, 0)

    pltpu.sync_copy(den_v, den_hbm.at[wid])


def _edge_weights(s, d, smax16, src, dst):
    mesh = plsc.VectorSubcoreMesh(core_axis_name="c", subcore_axis_name="s")
    f = pl.kernel(
        _ew_body,
        out_type=(
            jax.ShapeDtypeStruct((E,), jnp.float32),
            jax.ShapeDtypeStruct((NUM_TILES, N), jnp.float32),
        ),
        mesh=mesh,
        scratch_types=(
            pltpu.VMEM((N,), jnp.float32),
            pltpu.VMEM((N,), jnp.float32),
            pltpu.VMEM((N,), jnp.float32),
            pltpu.VMEM((16,), jnp.float32),
            pltpu.VMEM((CHUNK,), jnp.int32),
            pltpu.VMEM((CHUNK,), jnp.int32),
            pltpu.VMEM((CHUNK,), jnp.float32),
        ),
        compiler_params=pltpu.CompilerParams(needs_layout_passes=False),
    )
    return f(s, d, smax16, src, dst)


def _rows_body(h_hbm, ea_hbm, src_hbm, dst_hbm,
               out_hbm,
               src_v, dst_v, ea_v, rows_v, out_sp, sem):
    c = lax.axis_index("c")
    sub = lax.axis_index("s")
    wid = c * 16 + sub
    base = wid * EPT

    zero16 = jnp.zeros((16,), jnp.float32)

    def _zrows(e, carry):
        for f in range(8):
            rows_v[e, pl.ds(f * 16, 16)] = zero16
        return carry
    lax.fori_loop(0, CHUNK, _zrows, 0)

    myrow = sub * (N // 16)
    for r in range(6):
        pltpu.sync_copy(rows_v, out_sp.at[pl.ds(myrow + r * CHUNK, CHUNK)])
    pltpu.sync_copy(rows_v.at[pl.ds(0, 16)],
                    out_sp.at[pl.ds(myrow + 6 * CHUNK, 16)])

    plsc.subcore_barrier()

    def _chunk(k, carry):
        off = base + k * CHUNK
        pltpu.sync_copy(src_hbm.at[pl.ds(off, CHUNK)], src_v)
        pltpu.sync_copy(dst_hbm.at[pl.ds(off, CHUNK)], dst_v)
        pltpu.sync_copy(ea_hbm.at[pl.ds(off, CHUNK)], ea_v)
        pltpu.async_copy(h_hbm.at[src_v], rows_v, sem).wait()

        def _scale(e, carry2):
            idx = lax.broadcast(e, (16,))
            sca = plsc.load_gather(ea_v, [idx])
            for f in range(8):
                rows_v[e, pl.ds(f * 16, 16)] = rows_v[e, pl.ds(f * 16, 16)] * sca
            return carry2
        lax.fori_loop(0, CHUNK, _scale, 0)

        pltpu.sync_copy(rows_v, out_sp.at[dst_v], add=True)
        return carry

    lax.fori_loop(0, NCHUNK, _chunk, 0)

    plsc.subcore_barrier()

    pltpu.sync_copy(out_sp.at[pl.ds(myrow, N // 16)],
                    out_hbm.at[c, pl.ds(myrow, N // 16)])


def _edge_rows(h, ea, src, dst):
    mesh = plsc.VectorSubcoreMesh(core_axis_name="c", subcore_axis_name="s")
    f = pl.kernel(
        _rows_body,
        out_type=jax.ShapeDtypeStruct((2, N, LATENT), jnp.float32),
        mesh=mesh,
        scratch_types=(
            pltpu.VMEM((CHUNK,), jnp.int32),
            pltpu.VMEM((CHUNK,), jnp.int32),
            pltpu.VMEM((CHUNK,), jnp.float32),
            pltpu.VMEM((CHUNK, LATENT), jnp.float32),
            pltpu.VMEM_SHARED((N, LATENT), jnp.float32),
            pltpu.SemaphoreType.DMA,
        ),
        compiler_params=pltpu.CompilerParams(needs_layout_passes=False),
    )
    return f(h, ea, src, dst)


def kernel(z, edge_index, params):
    src = edge_index[0]
    dst = edge_index[1]
    x = _fc(z, params["fc_W"], params["fc_b"]).reshape(N, LATENT)
    convs = params["convs"]
    for i, p in enumerate(convs):
        h, s, d, smax = _pre(x, p["W"], p["a_src"], p["a_dst"])
        smax16 = jnp.broadcast_to(smax.reshape(1), (16,))
        ea, den = _edge_weights(s.reshape(N), d.reshape(N), smax16, src, dst)
        part = _edge_rows(h, ea, src, dst)
        x = _combine(part[0], part[1], den, h, s, d, smax,
                     p["b"], relu=(i < len(convs) - 1))
    return x


# speedup vs baseline: 23.7325x; 1.2792x over previous
import functools

import jax
import jax.numpy as jnp
from jax import lax
from jax.experimental import pallas as pl
from jax.experimental.pallas import tpu as pltpu
from jax.experimental.pallas import tpu_sc as plsc

B = 64
NUM_NODES = 196
N = B * NUM_NODES
E = 401408
LATENT = 128
NEG = 0.2

NUM_TILES = 32
EPT = E // NUM_TILES
CHUNK = 112
NCHUNK = EPT // CHUNK
ROWBLK = 896


def _leaky(x):
    return jnp.where(x > 0, x, NEG * x)


def _fc_body(z_ref, w_ref, b_ref, o_ref):
    acc = lax.dot_general(z_ref[...], w_ref[...],
                          (((1,), (1,)), ((), ())),
                          preferred_element_type=jnp.float32)
    o_ref[...] = jnp.maximum(acc + b_ref[...], 0.0)


def _fc(z, fc_w, fc_b):
    nblk = 49
    blk = (NUM_NODES * LATENT) // nblk
    return pl.pallas_call(
        _fc_body,
        grid=(nblk,),
        in_specs=[
            pl.BlockSpec((B, LATENT), lambda i: (0, 0)),
            pl.BlockSpec((blk, LATENT), lambda i: (i, 0)),
            pl.BlockSpec((1, blk), lambda i: (0, i)),
        ],
        out_specs=pl.BlockSpec((B, blk), lambda i: (0, i)),
        out_shape=jax.ShapeDtypeStruct((B, NUM_NODES * LATENT), jnp.float32),
    )(z, fc_w, fc_b.reshape(1, -1))


def _pre_body(x_ref, w_ref, asrc_ref, adst_ref, h_ref, s_ref, d_ref, sm_ref):
    i = pl.program_id(0)
    h = lax.dot_general(x_ref[...], w_ref[...],
                        (((1,), (0,)), ((), ())),
                        preferred_element_type=jnp.float32)
    h_ref[...] = h
    s = jnp.sum(h * asrc_ref[...], axis=1, keepdims=True)
    d = jnp.sum(h * adst_ref[...], axis=1, keepdims=True)
    s_ref[...] = s
    d_ref[...] = d

    @pl.when(i == 0)
    def _():
        sm_ref[...] = jnp.full((1, 1), -jnp.inf, jnp.float32)

    sm_ref[...] = jnp.maximum(sm_ref[...], jnp.max(s))


def _pre(x, w, a_src, a_dst):
    nblk = N // ROWBLK
    return pl.pallas_call(
        _pre_body,
        grid=(nblk,),
        in_specs=[
            pl.BlockSpec((ROWBLK, LATENT), lambda i: (i, 0)),
            pl.BlockSpec((LATENT, LATENT), lambda i: (0, 0)),
            pl.BlockSpec((1, LATENT), lambda i: (0, 0)),
            pl.BlockSpec((1, LATENT), lambda i: (0, 0)),
        ],
        out_specs=[
            pl.BlockSpec((ROWBLK, LATENT), lambda i: (i, 0)),
            pl.BlockSpec((ROWBLK, 1), lambda i: (i, 0)),
            pl.BlockSpec((ROWBLK, 1), lambda i: (i, 0)),
            pl.BlockSpec((1, 1), lambda i: (0, 0)),
        ],
        out_shape=[
            jax.ShapeDtypeStruct((N, LATENT), jnp.float32),
            jax.ShapeDtypeStruct((N, 1), jnp.float32),
            jax.ShapeDtypeStruct((N, 1), jnp.float32),
            jax.ShapeDtypeStruct((1, 1), jnp.float32),
        ],
    )(x, w, a_src.reshape(1, -1), a_dst.reshape(1, -1))


def _combine_body(relu, p0_ref, p1_ref, den_ref, h_ref, s_ref, d_ref,
                  sm_ref, b_ref, o_ref):
    s = s_ref[...]
    d = d_ref[...]
    smax = sm_ref[0, 0]
    m = _leaky(smax + d)
    ea_self = jnp.exp(_leaky(s + d) - m)
    den_tot = jnp.sum(den_ref[...], axis=0)[:, None] + ea_self
    num = p0_ref[...] + p1_ref[...] + ea_self * h_ref[...]
    out = num / den_tot + b_ref[...]
    if relu:
        out = jnp.maximum(out, 0.0)
    o_ref[...] = out


def _combine(p0, p1, den, h, s, d, smax, bias, relu):
    nblk = N // ROWBLK
    return pl.pallas_call(
        functools.partial(_combine_body, relu),
        grid=(nblk,),
        in_specs=[
            pl.BlockSpec((ROWBLK, LATENT), lambda i: (i, 0)),
            pl.BlockSpec((ROWBLK, LATENT), lambda i: (i, 0)),
            pl.BlockSpec((NUM_TILES, ROWBLK), lambda i: (0, i)),
            pl.BlockSpec((ROWBLK, LATENT), lambda i: (i, 0)),
            pl.BlockSpec((ROWBLK, 1), lambda i: (i, 0)),
            pl.BlockSpec((ROWBLK, 1), lambda i: (i, 0)),
            pl.BlockSpec((1, 1), lambda i: (0, 0)),
            pl.BlockSpec((1, LATENT), lambda i: (0, 0)),
        ],
        out_specs=pl.BlockSpec((ROWBLK, LATENT), lambda i: (i, 0)),
        out_shape=jax.ShapeDtypeStruct((N, LATENT), jnp.float32),
    )(p0, p1, den, h, s, d, smax, bias.reshape(1, -1))


def _ew_body(s_hbm, d_hbm, sm_hbm, src_hbm, dst_hbm,
             ea_hbm, den_hbm,
             s_v, d_v, den_v, sm_v, src_all, dst_all, ea_all):
    c = lax.axis_index("c")
    sub = lax.axis_index("s")
    wid = c * 16 + sub
    base = wid * EPT

    pltpu.sync_copy(s_hbm, s_v)
    pltpu.sync_copy(d_hbm, d_v)
    pltpu.sync_copy(sm_hbm, sm_v)
    pltpu.sync_copy(src_hbm.at[pl.ds(base, EPT)], src_all)
    pltpu.sync_copy(dst_hbm.at[pl.ds(base, EPT)], dst_all)

    zero16 = jnp.zeros((16,), jnp.float32)

    def _zden(i, carry):
        den_v[pl.ds(i * 16, 16)] = zero16
        return carry
    lax.fori_loop(0, N // 16, _zden, 0)

    smax = sm_v[...]

    UNROLL = 4
    def _group(i, carry):
        for u in range(UNROLL):
            o = (i * UNROLL + u) * 16
            isrc = src_all[pl.ds(o, 16)]
            idst = dst_all[pl.ds(o, 16)]
            sv = plsc.load_gather(s_v, [isrc])
            dv = plsc.load_gather(d_v, [idst])
            t = smax + dv
            m = jnp.where(t > 0, t, NEG * t)
            a = sv + dv
            a = jnp.where(a > 0, a, NEG * a)
            ea = jnp.exp(a - m)
            plsc.addupdate_scatter(den_v, [idst], ea)
            ea_all[pl.ds(o, 16)] = ea
        return carry

    lax.fori_loop(0, EPT // (16 * UNROLL), _group, 0)

    pltpu.sync_copy(ea_all, ea_hbm.at[pl.ds(base, EPT)])
    pltpu.sync_copy(den_v, den_hbm.at[wid])


def _edge_weights(s, d, smax16, src, dst):
    mesh = plsc.VectorSubcoreMesh(core_axis_name="c", subcore_axis_name="s")
    f = pl.kernel(
        _ew_body,
        out_type=(
            jax.ShapeDtypeStruct((E,), jnp.float32),
            jax.ShapeDtypeStruct((NUM_TILES, N), jnp.float32),
        ),
        mesh=mesh,
        scratch_types=(
            pltpu.VMEM((N,), jnp.float32),
            pltpu.VMEM((N,), jnp.float32),
            pltpu.VMEM((N,), jnp.float32),
            pltpu.VMEM((16,), jnp.float32),
            pltpu.VMEM((EPT,), jnp.int32),
            pltpu.VMEM((EPT,), jnp.int32),
            pltpu.VMEM((EPT,), jnp.float32),
        ),
        compiler_params=pltpu.CompilerParams(needs_layout_passes=False),
    )
    return f(s, d, smax16, src, dst)


def _rows_body(h_hbm, ea_hbm, src_hbm, dst_hbm,
               out_hbm,
               src_a, dst_a, ea_a, rows_a,
               src_b, dst_b, ea_b, rows_b,
               out_sp, gsem, ssem_a, ssem_b):
    c = lax.axis_index("c")
    sub = lax.axis_index("s")
    wid = c * 16 + sub
    base = wid * EPT

    zero16 = jnp.zeros((16,), jnp.float32)

    def _zrows(e, carry):
        for f in range(8):
            rows_a[e, pl.ds(f * 16, 16)] = zero16
        return carry
    lax.fori_loop(0, CHUNK, _zrows, 0)

    myrow = sub * (N // 16)
    nz = (N // 16) // CHUNK
    for r in range(nz):
        pltpu.sync_copy(rows_a, out_sp.at[pl.ds(myrow + r * CHUNK, CHUNK)])
    rem = (N // 16) - nz * CHUNK
    if rem:
        pltpu.sync_copy(rows_a.at[pl.ds(0, rem)],
                        out_sp.at[pl.ds(myrow + nz * CHUNK, rem)])

    plsc.subcore_barrier()

    bufs = ((src_a, dst_a, ea_a, rows_a, ssem_a),
            (src_b, dst_b, ea_b, rows_b, ssem_b))

    def _pair(k2, carry):
        for half, (srcv, dstv, eav, rowsv, ssem) in enumerate(bufs):
            k = k2 * 2 + half
            off = base + k * CHUNK

            @pl.when(k2 > 0)
            def _():
                pltpu.make_async_copy(rowsv, out_sp.at[dstv], ssem).wait()

            pltpu.sync_copy(src_hbm.at[pl.ds(off, CHUNK)], srcv)
            pltpu.sync_copy(dst_hbm.at[pl.ds(off, CHUNK)], dstv)
            pltpu.sync_copy(ea_hbm.at[pl.ds(off, CHUNK)], eav)
            pltpu.async_copy(h_hbm.at[srcv], rowsv, gsem).wait()

            def _scale(e2, carry2):
                for u in range(2):
                    e = e2 * 2 + u
                    idx = lax.broadcast(e, (16,))
                    sca = plsc.load_gather(eav, [idx])
                    for f in range(8):
                        rowsv[e, pl.ds(f * 16, 16)] = (
                            rowsv[e, pl.ds(f * 16, 16)] * sca)
                return carry2
            lax.fori_loop(0, CHUNK // 2, _scale, 0)

            pltpu.async_copy(rowsv, out_sp.at[dstv], ssem, add=True)
        return carry

    lax.fori_loop(0, NCHUNK // 2, _pair, 0)

    pltpu.make_async_copy(rows_a, out_sp.at[dst_a], ssem_a).wait()
    pltpu.make_async_copy(rows_b, out_sp.at[dst_b], ssem_b).wait()

    plsc.subcore_barrier()

    pltpu.sync_copy(out_sp.at[pl.ds(myrow, N // 16)],
                    out_hbm.at[c, pl.ds(myrow, N // 16)])


def _edge_rows(h, ea, src, dst):
    mesh = plsc.VectorSubcoreMesh(core_axis_name="c", subcore_axis_name="s")
    f = pl.kernel(
        _rows_body,
        out_type=jax.ShapeDtypeStruct((2, N, LATENT), jnp.float32),
        mesh=mesh,
        scratch_types=(
            pltpu.VMEM((CHUNK,), jnp.int32),
            pltpu.VMEM((CHUNK,), jnp.int32),
            pltpu.VMEM((CHUNK,), jnp.float32),
            pltpu.VMEM((CHUNK, LATENT), jnp.float32),
            pltpu.VMEM((CHUNK,), jnp.int32),
            pltpu.VMEM((CHUNK,), jnp.int32),
            pltpu.VMEM((CHUNK,), jnp.float32),
            pltpu.VMEM((CHUNK, LATENT), jnp.float32),
            pltpu.VMEM_SHARED((N, LATENT), jnp.float32),
            pltpu.SemaphoreType.DMA,
            pltpu.SemaphoreType.DMA,
            pltpu.SemaphoreType.DMA,
        ),
        compiler_params=pltpu.CompilerParams(needs_layout_passes=False),
    )
    return f(h, ea, src, dst)


def kernel(z, edge_index, params):
    src = edge_index[0]
    dst = edge_index[1]
    x = _fc(z, params["fc_W"], params["fc_b"]).reshape(N, LATENT)
    convs = params["convs"]
    for i, p in enumerate(convs):
        h, s, d, smax = _pre(x, p["W"], p["a_src"], p["a_dst"])
        smax16 = jnp.broadcast_to(smax.reshape(1), (16,))
        ea, den = _edge_weights(s.reshape(N), d.reshape(N), smax16, src, dst)
        part = _edge_rows(h, ea, src, dst)
        x = _combine(part[0], part[1], den, h, s, d, smax,
                     p["b"], relu=(i < len(convs) - 1))
    return x
